# baseline (speedup 1.0000x reference)
import dataclasses

import jax
import jax.numpy as jnp
from jax import lax
from jax.experimental import pallas as pl
from jax.experimental.pallas import tpu as pltpu
from jax.experimental.pallas import tpu_sc as plsc

N = 100000
E = 1600000
G = 256
F = 33

NC = 2
NS = 16
NW = NC * NS

EPAD = 1638400
EPW = EPAD // NS
BLK = 1024
EB = EPAD // BLK
EBW = EPW // BLK
CHB = 20
NCH = EBW // CHB

HALF = 50000
ACC = 50048
TRASH = 50008
RW = ACC // NS
ZR = 391

NPADG = 102400
GB = NPADG // NW
GACC = 272

_MESH = plsc.VectorSubcoreMesh(core_axis_name="c", subcore_axis_name="s")

TR = 1000
GRID = N // TR


def _init_const(ref, rows, val):
    @pl.loop(0, rows)
    def _(i):
        ref[i, :] = jnp.full((16,), val, jnp.float32)


def _agg_body(tbl_hbm, gix_hbm, ldst_hbm, cpar_hbm, out_hbm,
              gidxc, didxc, cbuf, rows, zbuf, acc, gsem, ssem):
    c = lax.axis_index("c")
    s = lax.axis_index("s")

    pltpu.sync_copy(cpar_hbm, cbuf)
    cval = jnp.max(cbuf[0, :])
    _init_const(zbuf, ZR, 0.0)

    def do_block(b, q, drain):
        if drain:
            pltpu.make_async_copy(tbl_hbm.at[pl.ds(0, BLK)], rows.at[q],
                                  ssem).wait()
        pltpu.async_copy(tbl_hbm.at[gidxc.at[pl.ds(b * BLK, BLK)]],
                         rows.at[q], gsem).wait()
        pltpu.async_copy(rows.at[q], acc.at[didxc.at[pl.ds(b * BLK, BLK)]],
                         ssem, add=True)

    @pl.loop(0, cval)
    def _(col):
        for k in range(RW // ZR):
            pltpu.sync_copy(zbuf, acc.at[pl.ds(s * RW + k * ZR, ZR)])
        plsc.subcore_barrier()

        @pl.loop(0, NCH)
        def _(ch):
            @pl.when(ch > 0)
            def _():
                pltpu.make_async_copy(tbl_hbm.at[pl.ds(0, BLK)], rows.at[0],
                                      ssem).wait()
                pltpu.make_async_copy(tbl_hbm.at[pl.ds(0, BLK)], rows.at[1],
                                      ssem).wait()

            ibase = (s * EBW + ch * CHB) * BLK
            pltpu.sync_copy(gix_hbm.at[col, pl.ds(ibase, CHB * BLK)], gidxc)
            pltpu.sync_copy(ldst_hbm.at[c, pl.ds(ibase, CHB * BLK)], didxc)

            do_block(0, 0, drain=False)
            do_block(1, 1, drain=False)

            @pl.loop(2, CHB, step=2)
            def _(b):
                do_block(b, 0, drain=True)
                do_block(b + 1, 1, drain=True)

        pltpu.make_async_copy(tbl_hbm.at[pl.ds(0, BLK)], rows.at[0],
                              ssem).wait()
        pltpu.make_async_copy(tbl_hbm.at[pl.ds(0, BLK)], rows.at[1],
                              ssem).wait()
        plsc.subcore_barrier()

        obase = c * HALF + s * RW

        @pl.when(s < NS - 1)
        def _():
            pltpu.sync_copy(acc.at[pl.ds(s * RW, RW)],
                            out_hbm.at[col, pl.ds(obase, RW)])

        @pl.when(s == NS - 1)
        def _():
            pltpu.sync_copy(
                acc.at[pl.ds(s * RW, HALF - (NS - 1) * RW)],
                out_hbm.at[col, pl.ds(obase, HALF - (NS - 1) * RW)])


_SC_CP = pltpu.CompilerParams(use_tc_tiling_on_sc=False)
if "needs_layout_passes" in pltpu.CompilerParams.__dataclass_fields__:
    _SC_CP = dataclasses.replace(_SC_CP, needs_layout_passes=False)

_AGG = pl.kernel(
    _agg_body,
    out_type=jax.ShapeDtypeStruct((9, N, 16), jnp.float32),
    compiler_params=_SC_CP,
    mesh=_MESH,
    scratch_types=[
        pltpu.VMEM((CHB * BLK,), jnp.int32),
        pltpu.VMEM((CHB * BLK,), jnp.int32),
        pltpu.VMEM((1, 16), jnp.int32),
        pltpu.VMEM((2, BLK, 16), jnp.float32),
        pltpu.VMEM((ZR, 16), jnp.float32),
        pltpu.VMEM_SHARED((ACC, 16), jnp.float32),
        pltpu.SemaphoreType.DMA,
        pltpu.SemaphoreType.DMA,
    ],
)


def _gep_body(h_hbm, b_hbm, sums_hbm, counts_hbm, bidx, hbuf, zbuf, acc, ssem):
    c = lax.axis_index("c")
    s = lax.axis_index("s")
    wid = c * NS + s
    nr = GB // 128
    pltpu.sync_copy(b_hbm.at[c, pl.ds(wid * nr, nr)], bidx)
    _init_const(zbuf, GACC, 0.0)

    def scatter_all():
        cps = [
            pltpu.async_copy(hbuf.at[pl.ds(j * 128, 128)],
                             acc.at[bidx.at[j]], ssem, add=True)
            for j in range(nr)
        ]
        for cp in cps:
            cp.wait()

    _init_const(hbuf, GB, 1.0)

    @pl.when(s == 0)
    def _():
        pltpu.sync_copy(zbuf, acc)

    plsc.subcore_barrier()
    scatter_all()
    plsc.subcore_barrier()

    @pl.when(s == 0)
    def _():
        pltpu.sync_copy(acc, counts_hbm.at[c])

    @pl.loop(0, 9)
    def _(col):
        @pl.when(s == 0)
        def _():
            pltpu.sync_copy(zbuf, acc)

        plsc.subcore_barrier()
        pltpu.sync_copy(h_hbm.at[col, pl.ds(wid * GB, GB)], hbuf)
        scatter_all()
        plsc.subcore_barrier()

        @pl.when(s == 0)
        def _():
            pltpu.sync_copy(acc, sums_hbm.at[c, col])


_GEP = pl.kernel(
    _gep_body,
    out_type=[
        jax.ShapeDtypeStruct((NC, 9, GACC, 16), jnp.float32),
        jax.ShapeDtypeStruct((NC, GACC, 16), jnp.float32),
    ],
    compiler_params=_SC_CP,
    mesh=_MESH,
    scratch_types=[
        pltpu.VMEM((GB // 128, 128), jnp.int32),
        pltpu.VMEM((GB, 16), jnp.float32),
        pltpu.VMEM((GACC, 16), jnp.float32),
        pltpu.VMEM_SHARED((GACC, 16), jnp.float32),
        pltpu.SemaphoreType.DMA,
    ],
)


def _tc_index(src2, dst2):
    def bdy(s_ref, d_ref, g_ref, l_ref):
        sv = s_ref[...]
        dv = d_ref[...]
        for cc in range(9):
            g_ref[cc, :, :] = sv + cc * N
        for cc in range(NC):
            base = cc * HALF
            ok = (dv >= base) & (dv < base + HALF)
            l_ref[cc, :, :] = jnp.where(ok, dv - base, TRASH)

    eb = 16
    return pl.pallas_call(
        bdy,
        grid=(EB // eb,),
        in_specs=[
            pl.BlockSpec((eb, BLK), lambda i: (i, 0)),
            pl.BlockSpec((eb, BLK), lambda i: (i, 0)),
        ],
        out_specs=[
            pl.BlockSpec((9, eb, BLK), lambda i: (0, i, 0)),
            pl.BlockSpec((NC, eb, BLK), lambda i: (0, i, 0)),
        ],
        out_shape=[
            jax.ShapeDtypeStruct((9, EB, BLK), jnp.int32),
            jax.ShapeDtypeStruct((NC, EB, BLK), jnp.int32),
        ],
    )(src2, dst2)


def _tc_first(degm, x, w1p):
    def bdy(deg_ref, x_ref, w_ref, z_ref, g_ref, dinv_ref, d2_ref):
        deg = deg_ref[0, :, 0:1] + 1.0
        dinv = lax.rsqrt(deg)
        z = jnp.dot(x_ref[...], w_ref[...], preferred_element_type=jnp.float32)
        g = z * dinv
        z_ref[...] = z
        for cc in range(3):
            g_ref[cc, :, :] = g[:, cc * 16:(cc + 1) * 16]
        dinv_ref[...] = jnp.broadcast_to(dinv, (TR, 8))
        d2_ref[...] = jnp.broadcast_to(dinv * dinv, (TR, 8))

    return pl.pallas_call(
        bdy,
        grid=(GRID,),
        in_specs=[
            pl.BlockSpec((1, TR, 16), lambda i: (0, i, 0)),
            pl.BlockSpec((TR, F), lambda i: (i, 0)),
            pl.BlockSpec((F, 48), lambda i: (0, 0)),
        ],
        out_specs=[
            pl.BlockSpec((TR, 48), lambda i: (i, 0)),
            pl.BlockSpec((9, TR, 16), lambda i: (0, i, 0)),
            pl.BlockSpec((TR, 8), lambda i: (i, 0)),
            pl.BlockSpec((TR, 8), lambda i: (i, 0)),
        ],
        out_shape=[
            jax.ShapeDtypeStruct((N, 48), jnp.float32),
            jax.ShapeDtypeStruct((9, N, 16), jnp.float32),
            jax.ShapeDtypeStruct((N, 8), jnp.float32),
            jax.ShapeDtypeStruct((N, 8), jnp.float32),
        ],
    )(degm, x, w1p)


def _tc_mid(din, dout, sv, z, dinv8, d28, bp, wp):
    cn = dout // 16

    def bdy(s_ref, z_ref, dinv_ref, d2_ref, b_ref, w_ref, zo_ref, g_ref):
        dinv = dinv_ref[:, 0:1]
        d2 = d2_ref[:, 0:1]
        sv = jnp.concatenate([s_ref[cc] for cc in range(din // 16)], axis=1)
        a = jnp.maximum(dinv * sv + d2 * z_ref[...] + b_ref[...], 0.0)
        z2 = jnp.dot(a, w_ref[...], preferred_element_type=jnp.float32)
        g = z2 * dinv
        zo_ref[...] = z2
        for cc in range(cn):
            g_ref[cc, :, :] = g[:, cc * 16:(cc + 1) * 16]

    return pl.pallas_call(
        bdy,
        grid=(GRID,),
        in_specs=[
            pl.BlockSpec((din // 16, TR, 16), lambda i: (0, i, 0)),
            pl.BlockSpec((TR, din), lambda i: (i, 0)),
            pl.BlockSpec((TR, 8), lambda i: (i, 0)),
            pl.BlockSpec((TR, 8), lambda i: (i, 0)),
            pl.BlockSpec((1, din), lambda i: (0, 0)),
            pl.BlockSpec((din, dout), lambda i: (0, 0)),
        ],
        out_specs=[
            pl.BlockSpec((TR, dout), lambda i: (i, 0)),
            pl.BlockSpec((9, TR, 16), lambda i: (0, i, 0)),
        ],
        out_shape=[
            jax.ShapeDtypeStruct((N, dout), jnp.float32),
            jax.ShapeDtypeStruct((9, N, 16), jnp.float32),
        ],
    )(sv, z, dinv8, d28, bp, wp)


def _tc_last(sv, z, dinv8, d28, bp, relu_last):
    def bdy(s_ref, z_ref, dinv_ref, d2_ref, b_ref, h_ref):
        dinv = dinv_ref[:, 0:1]
        d2 = d2_ref[:, 0:1]
        sv = jnp.concatenate([s_ref[cc] for cc in range(9)], axis=1)
        a = dinv * sv + d2 * z_ref[...] + b_ref[...]
        if relu_last:
            a = jnp.maximum(a, 0.0)
        for cc in range(9):
            h_ref[cc, :, :] = a[:, cc * 16:(cc + 1) * 16]

    return pl.pallas_call(
        bdy,
        grid=(GRID,),
        in_specs=[
            pl.BlockSpec((9, TR, 16), lambda i: (0, i, 0)),
            pl.BlockSpec((TR, 144), lambda i: (i, 0)),
            pl.BlockSpec((TR, 8), lambda i: (i, 0)),
            pl.BlockSpec((TR, 8), lambda i: (i, 0)),
            pl.BlockSpec((1, 144), lambda i: (0, 0)),
        ],
        out_specs=[pl.BlockSpec((9, TR, 16), lambda i: (0, i, 0))],
        out_shape=[jax.ShapeDtypeStruct((9, NPADG, 16), jnp.float32)],
    )(sv, z, dinv8, d28, bp)[0]


def _tc_head(s1, c1, s2, c2, w14, b14, w15, b15, w24, b24, w25, b25,
             fc1w, fc1b, fc2w, fc2b, outw, outb):
    def bdy(s1_ref, c1_ref, s2_ref, c2_ref, w14_ref, b14_ref, w15_ref, b15_ref,
            w24_ref, b24_ref, w25_ref, b25_ref, fc1w_ref, fc1b_ref,
            fc2w_ref, fc2b_ref, outw_ref, outb_ref, o_ref):
        def pool(sref, cref):
            svv = jnp.concatenate(
                [sref[0, cc] + sref[1, cc] for cc in range(9)], axis=1)
            cvv = cref[0, :, 0:1] + cref[1, :, 0:1]
            return (svv / jnp.maximum(cvv, 1.0))[:G]

        p1 = pool(s1_ref, c1_ref)
        p2 = pool(s2_ref, c2_ref)
        x = jnp.maximum(jnp.dot(p1, w14_ref[...],
                                preferred_element_type=jnp.float32)
                        + b14_ref[...], 0.0)
        x = jnp.dot(x, w15_ref[...],
                    preferred_element_type=jnp.float32) + b15_ref[...]
        y = jnp.maximum(jnp.dot(p2, w24_ref[...],
                                preferred_element_type=jnp.float32)
                        + b24_ref[...], 0.0)
        y = jnp.dot(y, w25_ref[...],
                    preferred_element_type=jnp.float32) + b25_ref[...]
        xc = jnp.concatenate([x, y], axis=1)
        xc = jnp.maximum(jnp.dot(xc, fc1w_ref[...],
                                 preferred_element_type=jnp.float32)
                         + fc1b_ref[...], 0.0)
        xc = jnp.maximum(jnp.dot(xc, fc2w_ref[...],
                                 preferred_element_type=jnp.float32)
                         + fc2b_ref[...], 0.0)
        o_ref[...] = jax.nn.sigmoid(
            jnp.dot(xc, outw_ref[...], preferred_element_type=jnp.float32)
            + outb_ref[...])

    return pl.pallas_call(
        bdy,
        out_shape=jax.ShapeDtypeStruct((G, 1), jnp.float32),
    )(s1, c1, s2, c2, w14, b14, w15, b15, w24, b24, w25, b25,
      fc1w, fc1b, fc2w, fc2b, outw, outb)


def _padw(w, r, c):
    return jnp.pad(w, ((0, r - w.shape[0]), (0, c - w.shape[1])))


def _padb(b, d):
    return jnp.pad(b, (0, d - b.shape[0])).reshape(1, d)


def _cpar(v):
    return jnp.full((1, 16), v, jnp.int32)


def _tower(x, ei, batch, ones_tbl, w1, b1, w2, b2, w3, b3, relu_last):
    src2 = jnp.pad(ei[0], (0, EPAD - E)).reshape(EB, BLK)
    dst2 = jnp.pad(ei[1], (0, EPAD - E), constant_values=N).reshape(EB, BLK)
    bpad = jnp.pad(batch, (0, NPADG - N),
                   constant_values=G).reshape(NPADG // 128, 128)
    batch2 = jnp.stack([bpad, bpad])

    gix, ldst = _tc_index(src2, dst2)
    gix = gix.reshape(9, EPAD)
    ldst = ldst.reshape(NC, EPAD)

    w1p = _padw(w1, F, 48)
    w2p = _padw(w2, 48, 80)
    w3p = _padw(w3, 80, 144)
    b1p = _padb(b1, 48)
    b2p = _padb(b2, 80)
    b3p = _padb(b3, 144)

    degm = _AGG(ones_tbl, gix, ldst, _cpar(1))
    z1, g1, dinv8, d28 = _tc_first(degm, x, w1p)
    s1 = _AGG(g1.reshape(9 * N, 16), gix, ldst, _cpar(3))
    z2, g2 = _tc_mid(48, 80, s1, z1, dinv8, d28, b1p, w2p)
    s2 = _AGG(g2.reshape(9 * N, 16), gix, ldst, _cpar(5))
    z3, g3 = _tc_mid(80, 144, s2, z2, dinv8, d28, b2p, w3p)
    s3 = _AGG(g3.reshape(9 * N, 16), gix, ldst, _cpar(9))
    h4cb = _tc_last(s3, z3, dinv8, d28, b3p, relu_last)
    return _GEP(h4cb, batch2)


def kernel(pro1_x, pro1_edge_index, pro1_batch, pro2_x, pro2_edge_index,
           pro2_batch, w1W1, w1B1, w1W2, w1B2, w1W3, w1B3, w1W4, w1B4,
           w1W5, w1B5, w2W1, w2B1, w2W2, w2B2, w2W3, w2B3, w2W4, w2B4,
           w2W5, w2B5, fc1W, fc1B, fc2W, fc2B, outW, outB):
    ones_tbl = jnp.ones((9 * N, 16), jnp.float32)
    s1, c1 = _tower(pro1_x, pro1_edge_index, pro1_batch, ones_tbl,
                    w1W1, w1B1, w1W2, w1B2, w1W3, w1B3, relu_last=False)
    s2, c2 = _tower(pro2_x, pro2_edge_index, pro2_batch, ones_tbl,
                    w2W1, w2B1, w2W2, w2B2, w2W3, w2B3, relu_last=True)
    return _tc_head(
        s1, c1, s2, c2,
        _padw(w1W4, 144, 1024), w1B4.reshape(1, -1), w1W5, w1B5.reshape(1, -1),
        _padw(w2W4, 144, 1024), w2B4.reshape(1, -1), w2W5, w2B5.reshape(1, -1),
        fc1W, fc1B.reshape(1, -1), fc2W, fc2B.reshape(1, -1),
        outW, outB.reshape(1, -1))


# speedup vs baseline: 1.0029x; 1.0029x over previous
import dataclasses

import jax
import jax.numpy as jnp
from jax import lax
from jax.experimental import pallas as pl
from jax.experimental.pallas import tpu as pltpu
from jax.experimental.pallas import tpu_sc as plsc

N = 100000
E = 1600000
G = 256
F = 33

NC = 2
NS = 16
NW = NC * NS

EPAD = 1638400
EPW = EPAD // NS
BLK = 640
EB = EPAD // BLK
EBW = EPW // BLK
CHB = 16
NCH = EBW // CHB

HALF = 50000
ACC = 50048
TRASH = 50008
RW = ACC // NS
ZR = 391

NPADG = 102400
GB = NPADG // NW
GACC = 272

_MESH = plsc.VectorSubcoreMesh(core_axis_name="c", subcore_axis_name="s")

TR = 1000
GRID = N // TR


def _init_const(ref, rows, val):
    @pl.loop(0, rows)
    def _(i):
        ref[i, :] = jnp.full((16,), val, jnp.float32)


def _agg_body(tbl_hbm, gix_hbm, ldst_hbm, cpar_hbm, out_hbm,
              gidxc, didxc, cbuf, rows, zbuf, acc, gsem, ssem):
    c = lax.axis_index("c")
    s = lax.axis_index("s")

    pltpu.sync_copy(cpar_hbm, cbuf)
    cval = jnp.max(cbuf[0, :])
    _init_const(zbuf, ZR, 0.0)

    def fire_gather(b, q):
        pltpu.async_copy(tbl_hbm.at[gidxc.at[pl.ds(b * BLK, BLK)]],
                         rows.at[q], gsem.at[q])

    def fire_scatter(b, q):
        pltpu.async_copy(rows.at[q], acc.at[didxc.at[pl.ds(b * BLK, BLK)]],
                         ssem.at[q], add=True)

    def drain(sem, q):
        pltpu.make_async_copy(tbl_hbm.at[pl.ds(0, BLK)], rows.at[q],
                              sem.at[q]).wait()

    @pl.loop(0, cval)
    def _(col):
        for k in range(RW // ZR):
            pltpu.sync_copy(zbuf, acc.at[pl.ds(s * RW + k * ZR, ZR)])
        plsc.subcore_barrier()

        @pl.loop(0, NCH)
        def _(ch):
            @pl.when(ch > 0)
            def _():
                drain(ssem, (CHB - 2) % 4)
                drain(ssem, (CHB - 1) % 4)

            ibase = (s * EBW + ch * CHB) * BLK
            pltpu.sync_copy(gix_hbm.at[col, pl.ds(ibase, CHB * BLK)], gidxc)
            pltpu.sync_copy(ldst_hbm.at[c, pl.ds(ibase, CHB * BLK)], didxc)

            for b in range(CHB):
                q = b % 4
                if b >= 2:
                    drain(ssem, (b - 2) % 4)
                if b == 0:
                    fire_gather(0, 0)
                    fire_gather(1, 1)
                if b + 2 < CHB:
                    fire_gather(b + 2, (b + 2) % 4)
                drain(gsem, q)
                fire_scatter(b, q)

        drain(ssem, (CHB - 2) % 4)
        drain(ssem, (CHB - 1) % 4)
        plsc.subcore_barrier()

        obase = c * HALF + s * RW

        @pl.when(s < NS - 1)
        def _():
            pltpu.sync_copy(acc.at[pl.ds(s * RW, RW)],
                            out_hbm.at[col, pl.ds(obase, RW)])

        @pl.when(s == NS - 1)
        def _():
            pltpu.sync_copy(
                acc.at[pl.ds(s * RW, HALF - (NS - 1) * RW)],
                out_hbm.at[col, pl.ds(obase, HALF - (NS - 1) * RW)])


_SC_CP = pltpu.CompilerParams(use_tc_tiling_on_sc=False)
if "needs_layout_passes" in pltpu.CompilerParams.__dataclass_fields__:
    _SC_CP = dataclasses.replace(_SC_CP, needs_layout_passes=False)

_AGG = pl.kernel(
    _agg_body,
    out_type=jax.ShapeDtypeStruct((9, N, 16), jnp.float32),
    compiler_params=_SC_CP,
    mesh=_MESH,
    scratch_types=[
        pltpu.VMEM((CHB * BLK,), jnp.int32),
        pltpu.VMEM((CHB * BLK,), jnp.int32),
        pltpu.VMEM((1, 16), jnp.int32),
        pltpu.VMEM((4, BLK, 16), jnp.float32),
        pltpu.VMEM((ZR, 16), jnp.float32),
        pltpu.VMEM_SHARED((ACC, 16), jnp.float32),
        pltpu.SemaphoreType.DMA((4,)),
        pltpu.SemaphoreType.DMA((4,)),
    ],
)


def _gep_body(h_hbm, b_hbm, sums_hbm, counts_hbm, bidx, hbuf, zbuf, acc, ssem):
    c = lax.axis_index("c")
    s = lax.axis_index("s")
    wid = c * NS + s
    nr = GB // 128
    pltpu.sync_copy(b_hbm.at[c, pl.ds(wid * nr, nr)], bidx)
    _init_const(zbuf, GACC, 0.0)

    def scatter_all():
        cps = [
            pltpu.async_copy(hbuf.at[pl.ds(j * 128, 128)],
                             acc.at[bidx.at[j]], ssem, add=True)
            for j in range(nr)
        ]
        for cp in cps:
            cp.wait()

    _init_const(hbuf, GB, 1.0)

    @pl.when(s == 0)
    def _():
        pltpu.sync_copy(zbuf, acc)

    plsc.subcore_barrier()
    scatter_all()
    plsc.subcore_barrier()

    @pl.when(s == 0)
    def _():
        pltpu.sync_copy(acc, counts_hbm.at[c])

    @pl.loop(0, 9)
    def _(col):
        @pl.when(s == 0)
        def _():
            pltpu.sync_copy(zbuf, acc)

        plsc.subcore_barrier()
        pltpu.sync_copy(h_hbm.at[col, pl.ds(wid * GB, GB)], hbuf)
        scatter_all()
        plsc.subcore_barrier()

        @pl.when(s == 0)
        def _():
            pltpu.sync_copy(acc, sums_hbm.at[c, col])


_GEP = pl.kernel(
    _gep_body,
    out_type=[
        jax.ShapeDtypeStruct((NC, 9, GACC, 16), jnp.float32),
        jax.ShapeDtypeStruct((NC, GACC, 16), jnp.float32),
    ],
    compiler_params=_SC_CP,
    mesh=_MESH,
    scratch_types=[
        pltpu.VMEM((GB // 128, 128), jnp.int32),
        pltpu.VMEM((GB, 16), jnp.float32),
        pltpu.VMEM((GACC, 16), jnp.float32),
        pltpu.VMEM_SHARED((GACC, 16), jnp.float32),
        pltpu.SemaphoreType.DMA,
    ],
)


def _tc_index(src2, dst2):
    def bdy(s_ref, d_ref, g_ref, l_ref):
        sv = s_ref[...]
        dv = d_ref[...]
        for cc in range(9):
            g_ref[cc, :, :] = sv + cc * N
        for cc in range(NC):
            base = cc * HALF
            ok = (dv >= base) & (dv < base + HALF)
            l_ref[cc, :, :] = jnp.where(ok, dv - base, TRASH)

    eb = 16
    return pl.pallas_call(
        bdy,
        grid=(EB // eb,),
        in_specs=[
            pl.BlockSpec((eb, BLK), lambda i: (i, 0)),
            pl.BlockSpec((eb, BLK), lambda i: (i, 0)),
        ],
        out_specs=[
            pl.BlockSpec((9, eb, BLK), lambda i: (0, i, 0)),
            pl.BlockSpec((NC, eb, BLK), lambda i: (0, i, 0)),
        ],
        out_shape=[
            jax.ShapeDtypeStruct((9, EB, BLK), jnp.int32),
            jax.ShapeDtypeStruct((NC, EB, BLK), jnp.int32),
        ],
    )(src2, dst2)


def _tc_first(degm, x, w1p):
    def bdy(deg_ref, x_ref, w_ref, z_ref, g_ref, dinv_ref, d2_ref):
        deg = deg_ref[0, :, 0:1] + 1.0
        dinv = lax.rsqrt(deg)
        z = jnp.dot(x_ref[...], w_ref[...], preferred_element_type=jnp.float32)
        g = z * dinv
        z_ref[...] = z
        for cc in range(3):
            g_ref[cc, :, :] = g[:, cc * 16:(cc + 1) * 16]
        dinv_ref[...] = jnp.broadcast_to(dinv, (TR, 8))
        d2_ref[...] = jnp.broadcast_to(dinv * dinv, (TR, 8))

    return pl.pallas_call(
        bdy,
        grid=(GRID,),
        in_specs=[
            pl.BlockSpec((1, TR, 16), lambda i: (0, i, 0)),
            pl.BlockSpec((TR, F), lambda i: (i, 0)),
            pl.BlockSpec((F, 48), lambda i: (0, 0)),
        ],
        out_specs=[
            pl.BlockSpec((TR, 48), lambda i: (i, 0)),
            pl.BlockSpec((9, TR, 16), lambda i: (0, i, 0)),
            pl.BlockSpec((TR, 8), lambda i: (i, 0)),
            pl.BlockSpec((TR, 8), lambda i: (i, 0)),
        ],
        out_shape=[
            jax.ShapeDtypeStruct((N, 48), jnp.float32),
            jax.ShapeDtypeStruct((9, N, 16), jnp.float32),
            jax.ShapeDtypeStruct((N, 8), jnp.float32),
            jax.ShapeDtypeStruct((N, 8), jnp.float32),
        ],
    )(degm, x, w1p)


def _tc_mid(din, dout, sv, z, dinv8, d28, bp, wp):
    cn = dout // 16

    def bdy(s_ref, z_ref, dinv_ref, d2_ref, b_ref, w_ref, zo_ref, g_ref):
        dinv = dinv_ref[:, 0:1]
        d2 = d2_ref[:, 0:1]
        sv = jnp.concatenate([s_ref[cc] for cc in range(din // 16)], axis=1)
        a = jnp.maximum(dinv * sv + d2 * z_ref[...] + b_ref[...], 0.0)
        z2 = jnp.dot(a, w_ref[...], preferred_element_type=jnp.float32)
        g = z2 * dinv
        zo_ref[...] = z2
        for cc in range(cn):
            g_ref[cc, :, :] = g[:, cc * 16:(cc + 1) * 16]

    return pl.pallas_call(
        bdy,
        grid=(GRID,),
        in_specs=[
            pl.BlockSpec((din // 16, TR, 16), lambda i: (0, i, 0)),
            pl.BlockSpec((TR, din), lambda i: (i, 0)),
            pl.BlockSpec((TR, 8), lambda i: (i, 0)),
            pl.BlockSpec((TR, 8), lambda i: (i, 0)),
            pl.BlockSpec((1, din), lambda i: (0, 0)),
            pl.BlockSpec((din, dout), lambda i: (0, 0)),
        ],
        out_specs=[
            pl.BlockSpec((TR, dout), lambda i: (i, 0)),
            pl.BlockSpec((9, TR, 16), lambda i: (0, i, 0)),
        ],
        out_shape=[
            jax.ShapeDtypeStruct((N, dout), jnp.float32),
            jax.ShapeDtypeStruct((9, N, 16), jnp.float32),
        ],
    )(sv, z, dinv8, d28, bp, wp)


def _tc_last(sv, z, dinv8, d28, bp, relu_last):
    def bdy(s_ref, z_ref, dinv_ref, d2_ref, b_ref, h_ref):
        dinv = dinv_ref[:, 0:1]
        d2 = d2_ref[:, 0:1]
        sv = jnp.concatenate([s_ref[cc] for cc in range(9)], axis=1)
        a = dinv * sv + d2 * z_ref[...] + b_ref[...]
        if relu_last:
            a = jnp.maximum(a, 0.0)
        for cc in range(9):
            h_ref[cc, :, :] = a[:, cc * 16:(cc + 1) * 16]

    return pl.pallas_call(
        bdy,
        grid=(GRID,),
        in_specs=[
            pl.BlockSpec((9, TR, 16), lambda i: (0, i, 0)),
            pl.BlockSpec((TR, 144), lambda i: (i, 0)),
            pl.BlockSpec((TR, 8), lambda i: (i, 0)),
            pl.BlockSpec((TR, 8), lambda i: (i, 0)),
            pl.BlockSpec((1, 144), lambda i: (0, 0)),
        ],
        out_specs=[pl.BlockSpec((9, TR, 16), lambda i: (0, i, 0))],
        out_shape=[jax.ShapeDtypeStruct((9, NPADG, 16), jnp.float32)],
    )(sv, z, dinv8, d28, bp)[0]


def _tc_head(s1, c1, s2, c2, w14, b14, w15, b15, w24, b24, w25, b25,
             fc1w, fc1b, fc2w, fc2b, outw, outb):
    def bdy(s1_ref, c1_ref, s2_ref, c2_ref, w14_ref, b14_ref, w15_ref, b15_ref,
            w24_ref, b24_ref, w25_ref, b25_ref, fc1w_ref, fc1b_ref,
            fc2w_ref, fc2b_ref, outw_ref, outb_ref, o_ref):
        def pool(sref, cref):
            svv = jnp.concatenate(
                [sref[0, cc] + sref[1, cc] for cc in range(9)], axis=1)
            cvv = cref[0, :, 0:1] + cref[1, :, 0:1]
            return (svv / jnp.maximum(cvv, 1.0))[:G]

        p1 = pool(s1_ref, c1_ref)
        p2 = pool(s2_ref, c2_ref)
        x = jnp.maximum(jnp.dot(p1, w14_ref[...],
                                preferred_element_type=jnp.float32)
                        + b14_ref[...], 0.0)
        x = jnp.dot(x, w15_ref[...],
                    preferred_element_type=jnp.float32) + b15_ref[...]
        y = jnp.maximum(jnp.dot(p2, w24_ref[...],
                                preferred_element_type=jnp.float32)
                        + b24_ref[...], 0.0)
        y = jnp.dot(y, w25_ref[...],
                    preferred_element_type=jnp.float32) + b25_ref[...]
        xc = jnp.concatenate([x, y], axis=1)
        xc = jnp.maximum(jnp.dot(xc, fc1w_ref[...],
                                 preferred_element_type=jnp.float32)
                         + fc1b_ref[...], 0.0)
        xc = jnp.maximum(jnp.dot(xc, fc2w_ref[...],
                                 preferred_element_type=jnp.float32)
                         + fc2b_ref[...], 0.0)
        o_ref[...] = jax.nn.sigmoid(
            jnp.dot(xc, outw_ref[...], preferred_element_type=jnp.float32)
            + outb_ref[...])

    return pl.pallas_call(
        bdy,
        out_shape=jax.ShapeDtypeStruct((G, 1), jnp.float32),
    )(s1, c1, s2, c2, w14, b14, w15, b15, w24, b24, w25, b25,
      fc1w, fc1b, fc2w, fc2b, outw, outb)


def _padw(w, r, c):
    return jnp.pad(w, ((0, r - w.shape[0]), (0, c - w.shape[1])))


def _padb(b, d):
    return jnp.pad(b, (0, d - b.shape[0])).reshape(1, d)


def _cpar(v):
    return jnp.full((1, 16), v, jnp.int32)


def _tower(x, ei, batch, ones_tbl, w1, b1, w2, b2, w3, b3, relu_last):
    src2 = jnp.pad(ei[0], (0, EPAD - E)).reshape(EB, BLK)
    dst2 = jnp.pad(ei[1], (0, EPAD - E), constant_values=N).reshape(EB, BLK)
    bpad = jnp.pad(batch, (0, NPADG - N),
                   constant_values=G).reshape(NPADG // 128, 128)
    batch2 = jnp.stack([bpad, bpad])

    gix, ldst = _tc_index(src2, dst2)
    gix = gix.reshape(9, EPAD)
    ldst = ldst.reshape(NC, EPAD)

    w1p = _padw(w1, F, 48)
    w2p = _padw(w2, 48, 80)
    w3p = _padw(w3, 80, 144)
    b1p = _padb(b1, 48)
    b2p = _padb(b2, 80)
    b3p = _padb(b3, 144)

    degm = _AGG(ones_tbl, gix, ldst, _cpar(1))
    z1, g1, dinv8, d28 = _tc_first(degm, x, w1p)
    s1 = _AGG(g1.reshape(9 * N, 16), gix, ldst, _cpar(3))
    z2, g2 = _tc_mid(48, 80, s1, z1, dinv8, d28, b1p, w2p)
    s2 = _AGG(g2.reshape(9 * N, 16), gix, ldst, _cpar(5))
    z3, g3 = _tc_mid(80, 144, s2, z2, dinv8, d28, b2p, w3p)
    s3 = _AGG(g3.reshape(9 * N, 16), gix, ldst, _cpar(9))
    h4cb = _tc_last(s3, z3, dinv8, d28, b3p, relu_last)
    return _GEP(h4cb, batch2)


def kernel(pro1_x, pro1_edge_index, pro1_batch, pro2_x, pro2_edge_index,
           pro2_batch, w1W1, w1B1, w1W2, w1B2, w1W3, w1B3, w1W4, w1B4,
           w1W5, w1B5, w2W1, w2B1, w2W2, w2B2, w2W3, w2B3, w2W4, w2B4,
           w2W5, w2B5, fc1W, fc1B, fc2W, fc2B, outW, outB):
    ones_tbl = jnp.ones((9 * N, 16), jnp.float32)
    s1, c1 = _tower(pro1_x, pro1_edge_index, pro1_batch, ones_tbl,
                    w1W1, w1B1, w1W2, w1B2, w1W3, w1B3, relu_last=False)
    s2, c2 = _tower(pro2_x, pro2_edge_index, pro2_batch, ones_tbl,
                    w2W1, w2B1, w2W2, w2B2, w2W3, w2B3, relu_last=True)
    return _tc_head(
        s1, c1, s2, c2,
        _padw(w1W4, 144, 1024), w1B4.reshape(1, -1), w1W5, w1B5.reshape(1, -1),
        _padw(w2W4, 144, 1024), w2B4.reshape(1, -1), w2W5, w2B5.reshape(1, -1),
        fc1W, fc1B.reshape(1, -1), fc2W, fc2B.reshape(1, -1),
        outW, outB.reshape(1, -1))


# speedup vs baseline: 2.1464x; 2.1402x over previous
import dataclasses

import jax
import jax.numpy as jnp
from jax import lax
from jax.experimental import pallas as pl
from jax.experimental.pallas import tpu as pltpu
from jax.experimental.pallas import tpu_sc as plsc

N = 100000
E = 1600000
G = 256
F = 33

NC = 2
NS = 16
NW = NC * NS

EPAD = 1638400
EPW = EPAD // NW
BLK = 512
EB = EPAD // BLK
EBW = EPW // BLK
CHB = 10
NCH = EBW // CHB

ACC = 100352
RW = ACC // NS
ZR = 196

NPADG = 102400
GB = NPADG // NW
GACC = 272

_MESH = plsc.VectorSubcoreMesh(core_axis_name="c", subcore_axis_name="s")

TR = 1000
GRID = N // TR


def _init_const(ref, rows, val):
    @pl.loop(0, rows)
    def _(i):
        ref[i, :] = jnp.full((16,), val, jnp.float32)


def _agg_body(tbl_hbm, gix_hbm, dst_hbm, cpar_hbm, out_hbm,
              gidxc, didxc, cbuf, rows, zbuf, acc, gsem, ssem):
    c = lax.axis_index("c")
    s = lax.axis_index("s")
    wid = c * NS + s

    pltpu.sync_copy(cpar_hbm, cbuf)
    cval = jnp.max(cbuf[0, :])
    _init_const(zbuf, ZR, 0.0)

    def fire_gather(b, q):
        pltpu.async_copy(tbl_hbm.at[gidxc.at[pl.ds(b * BLK, BLK)]],
                         rows.at[q], gsem.at[q])

    def fire_scatter(b, q):
        pltpu.async_copy(rows.at[q], acc.at[didxc.at[pl.ds(b * BLK, BLK)]],
                         ssem.at[q], add=True)

    def drain(sem, q):
        pltpu.make_async_copy(tbl_hbm.at[pl.ds(0, BLK)], rows.at[q],
                              sem.at[q]).wait()

    @pl.loop(0, cval)
    def _(col):
        for k in range(RW // ZR):
            pltpu.sync_copy(zbuf, acc.at[pl.ds(s * RW + k * ZR, ZR)])
        plsc.subcore_barrier()

        @pl.loop(0, NCH)
        def _(ch):
            @pl.when(ch > 0)
            def _():
                drain(ssem, (CHB - 2) % 2)
                drain(ssem, (CHB - 1) % 2)

            ibase = (wid * EBW + ch * CHB) * BLK
            pltpu.sync_copy(gix_hbm.at[col, pl.ds(ibase, CHB * BLK)], gidxc)
            pltpu.sync_copy(dst_hbm.at[pl.ds(ibase, CHB * BLK)], didxc)

            for b in range(CHB):
                q = b % 2
                if b >= 2:
                    drain(ssem, q)
                fire_gather(b, q)
                drain(gsem, q)
                fire_scatter(b, q)

        drain(ssem, (CHB - 2) % 2)
        drain(ssem, (CHB - 1) % 2)
        plsc.subcore_barrier()

        pltpu.sync_copy(acc.at[pl.ds(s * RW, RW)],
                        out_hbm.at[c, col, pl.ds(s * RW, RW)])


_SC_CP = pltpu.CompilerParams(use_tc_tiling_on_sc=False)
if "needs_layout_passes" in pltpu.CompilerParams.__dataclass_fields__:
    _SC_CP = dataclasses.replace(_SC_CP, needs_layout_passes=False)

_AGG = pl.kernel(
    _agg_body,
    out_type=jax.ShapeDtypeStruct((NC, 9, ACC, 16), jnp.float32),
    compiler_params=_SC_CP,
    mesh=_MESH,
    scratch_types=[
        pltpu.VMEM((CHB * BLK,), jnp.int32),
        pltpu.VMEM((CHB * BLK,), jnp.int32),
        pltpu.VMEM((1, 16), jnp.int32),
        pltpu.VMEM((2, BLK, 16), jnp.float32),
        pltpu.VMEM((ZR, 16), jnp.float32),
        pltpu.VMEM_SHARED((ACC, 16), jnp.float32),
        pltpu.SemaphoreType.DMA((2,)),
        pltpu.SemaphoreType.DMA((2,)),
    ],
)


def _gep_body(h_hbm, b_hbm, sums_hbm, counts_hbm, bidx, hbuf, zbuf, acc, ssem):
    c = lax.axis_index("c")
    s = lax.axis_index("s")
    wid = c * NS + s
    nr = GB // 128
    pltpu.sync_copy(b_hbm.at[c, pl.ds(wid * nr, nr)], bidx)
    _init_const(zbuf, GACC, 0.0)

    def scatter_all():
        cps = [
            pltpu.async_copy(hbuf.at[pl.ds(j * 128, 128)],
                             acc.at[bidx.at[j]], ssem, add=True)
            for j in range(nr)
        ]
        for cp in cps:
            cp.wait()

    _init_const(hbuf, GB, 1.0)

    @pl.when(s == 0)
    def _():
        pltpu.sync_copy(zbuf, acc)

    plsc.subcore_barrier()
    scatter_all()
    plsc.subcore_barrier()

    @pl.when(s == 0)
    def _():
        pltpu.sync_copy(acc, counts_hbm.at[c])

    @pl.loop(0, 9)
    def _(col):
        @pl.when(s == 0)
        def _():
            pltpu.sync_copy(zbuf, acc)

        plsc.subcore_barrier()
        pltpu.sync_copy(h_hbm.at[col, pl.ds(wid * GB, GB)], hbuf)
        scatter_all()
        plsc.subcore_barrier()

        @pl.when(s == 0)
        def _():
            pltpu.sync_copy(acc, sums_hbm.at[c, col])


_GEP = pl.kernel(
    _gep_body,
    out_type=[
        jax.ShapeDtypeStruct((NC, 9, GACC, 16), jnp.float32),
        jax.ShapeDtypeStruct((NC, GACC, 16), jnp.float32),
    ],
    compiler_params=_SC_CP,
    mesh=_MESH,
    scratch_types=[
        pltpu.VMEM((GB // 128, 128), jnp.int32),
        pltpu.VMEM((GB, 16), jnp.float32),
        pltpu.VMEM((GACC, 16), jnp.float32),
        pltpu.VMEM_SHARED((GACC, 16), jnp.float32),
        pltpu.SemaphoreType.DMA,
    ],
)


def _tc_index(src2):
    def bdy(s_ref, g_ref):
        sv = s_ref[...]
        for cc in range(9):
            g_ref[cc, :, :] = sv + cc * N

    eb = 16
    return pl.pallas_call(
        bdy,
        grid=(EB // eb,),
        in_specs=[pl.BlockSpec((eb, BLK), lambda i: (i, 0))],
        out_specs=[pl.BlockSpec((9, eb, BLK), lambda i: (0, i, 0))],
        out_shape=[jax.ShapeDtypeStruct((9, EB, BLK), jnp.int32)],
    )(src2)[0]


def _tc_first(degm, x, w1p):
    def bdy(deg_ref, x_ref, w_ref, z_ref, g_ref, dinv_ref, d2_ref):
        deg = deg_ref[0, 0, :, 0:1] + deg_ref[1, 0, :, 0:1] + 1.0
        dinv = lax.rsqrt(deg)
        z = jnp.dot(x_ref[...], w_ref[...], preferred_element_type=jnp.float32)
        g = z * dinv
        z_ref[...] = z
        for cc in range(3):
            g_ref[cc, :, :] = g[:, cc * 16:(cc + 1) * 16]
        dinv_ref[...] = jnp.broadcast_to(dinv, (TR, 8))
        d2_ref[...] = jnp.broadcast_to(dinv * dinv, (TR, 8))

    return pl.pallas_call(
        bdy,
        grid=(GRID,),
        in_specs=[
            pl.BlockSpec((NC, 1, TR, 16), lambda i: (0, 0, i, 0)),
            pl.BlockSpec((TR, F), lambda i: (i, 0)),
            pl.BlockSpec((F, 48), lambda i: (0, 0)),
        ],
        out_specs=[
            pl.BlockSpec((TR, 48), lambda i: (i, 0)),
            pl.BlockSpec((9, TR, 16), lambda i: (0, i, 0)),
            pl.BlockSpec((TR, 8), lambda i: (i, 0)),
            pl.BlockSpec((TR, 8), lambda i: (i, 0)),
        ],
        out_shape=[
            jax.ShapeDtypeStruct((N, 48), jnp.float32),
            jax.ShapeDtypeStruct((9, N, 16), jnp.float32),
            jax.ShapeDtypeStruct((N, 8), jnp.float32),
            jax.ShapeDtypeStruct((N, 8), jnp.float32),
        ],
    )(degm, x, w1p)


def _tc_mid(din, dout, sv, z, dinv8, d28, bp, wp):
    cn = dout // 16

    def bdy(s_ref, z_ref, dinv_ref, d2_ref, b_ref, w_ref, zo_ref, g_ref):
        dinv = dinv_ref[:, 0:1]
        d2 = d2_ref[:, 0:1]
        sv = jnp.concatenate([s_ref[0, cc] + s_ref[1, cc]
                              for cc in range(din // 16)], axis=1)
        a = jnp.maximum(dinv * sv + d2 * z_ref[...] + b_ref[...], 0.0)
        z2 = jnp.dot(a, w_ref[...], preferred_element_type=jnp.float32)
        g = z2 * dinv
        zo_ref[...] = z2
        for cc in range(cn):
            g_ref[cc, :, :] = g[:, cc * 16:(cc + 1) * 16]

    return pl.pallas_call(
        bdy,
        grid=(GRID,),
        in_specs=[
            pl.BlockSpec((NC, din // 16, TR, 16), lambda i: (0, 0, i, 0)),
            pl.BlockSpec((TR, din), lambda i: (i, 0)),
            pl.BlockSpec((TR, 8), lambda i: (i, 0)),
            pl.BlockSpec((TR, 8), lambda i: (i, 0)),
            pl.BlockSpec((1, din), lambda i: (0, 0)),
            pl.BlockSpec((din, dout), lambda i: (0, 0)),
        ],
        out_specs=[
            pl.BlockSpec((TR, dout), lambda i: (i, 0)),
            pl.BlockSpec((9, TR, 16), lambda i: (0, i, 0)),
        ],
        out_shape=[
            jax.ShapeDtypeStruct((N, dout), jnp.float32),
            jax.ShapeDtypeStruct((9, N, 16), jnp.float32),
        ],
    )(sv, z, dinv8, d28, bp, wp)


def _tc_last(sv, z, dinv8, d28, bp, relu_last):
    def bdy(s_ref, z_ref, dinv_ref, d2_ref, b_ref, h_ref):
        dinv = dinv_ref[:, 0:1]
        d2 = d2_ref[:, 0:1]
        sv = jnp.concatenate([s_ref[0, cc] + s_ref[1, cc]
                              for cc in range(9)], axis=1)
        a = dinv * sv + d2 * z_ref[...] + b_ref[...]
        if relu_last:
            a = jnp.maximum(a, 0.0)
        for cc in range(9):
            h_ref[cc, :, :] = a[:, cc * 16:(cc + 1) * 16]

    return pl.pallas_call(
        bdy,
        grid=(GRID,),
        in_specs=[
            pl.BlockSpec((NC, 9, TR, 16), lambda i: (0, 0, i, 0)),
            pl.BlockSpec((TR, 144), lambda i: (i, 0)),
            pl.BlockSpec((TR, 8), lambda i: (i, 0)),
            pl.BlockSpec((TR, 8), lambda i: (i, 0)),
            pl.BlockSpec((1, 144), lambda i: (0, 0)),
        ],
        out_specs=[pl.BlockSpec((9, TR, 16), lambda i: (0, i, 0))],
        out_shape=[jax.ShapeDtypeStruct((9, NPADG, 16), jnp.float32)],
    )(sv, z, dinv8, d28, bp)[0]


def _tc_head(s1, c1, s2, c2, w14, b14, w15, b15, w24, b24, w25, b25,
             fc1w, fc1b, fc2w, fc2b, outw, outb):
    def bdy(s1_ref, c1_ref, s2_ref, c2_ref, w14_ref, b14_ref, w15_ref, b15_ref,
            w24_ref, b24_ref, w25_ref, b25_ref, fc1w_ref, fc1b_ref,
            fc2w_ref, fc2b_ref, outw_ref, outb_ref, o_ref):
        def pool(sref, cref):
            svv = jnp.concatenate(
                [sref[0, cc] + sref[1, cc] for cc in range(9)], axis=1)
            cvv = cref[0, :, 0:1] + cref[1, :, 0:1]
            return (svv / jnp.maximum(cvv, 1.0))[:G]

        p1 = pool(s1_ref, c1_ref)
        p2 = pool(s2_ref, c2_ref)
        x = jnp.maximum(jnp.dot(p1, w14_ref[...],
                                preferred_element_type=jnp.float32)
                        + b14_ref[...], 0.0)
        x = jnp.dot(x, w15_ref[...],
                    preferred_element_type=jnp.float32) + b15_ref[...]
        y = jnp.maximum(jnp.dot(p2, w24_ref[...],
                                preferred_element_type=jnp.float32)
                        + b24_ref[...], 0.0)
        y = jnp.dot(y, w25_ref[...],
                    preferred_element_type=jnp.float32) + b25_ref[...]
        xc = jnp.concatenate([x, y], axis=1)
        xc = jnp.maximum(jnp.dot(xc, fc1w_ref[...],
                                 preferred_element_type=jnp.float32)
                         + fc1b_ref[...], 0.0)
        xc = jnp.maximum(jnp.dot(xc, fc2w_ref[...],
                                 preferred_element_type=jnp.float32)
                         + fc2b_ref[...], 0.0)
        o_ref[...] = jax.nn.sigmoid(
            jnp.dot(xc, outw_ref[...], preferred_element_type=jnp.float32)
            + outb_ref[...])

    return pl.pallas_call(
        bdy,
        out_shape=jax.ShapeDtypeStruct((G, 1), jnp.float32),
    )(s1, c1, s2, c2, w14, b14, w15, b15, w24, b24, w25, b25,
      fc1w, fc1b, fc2w, fc2b, outw, outb)


def _padw(w, r, c):
    return jnp.pad(w, ((0, r - w.shape[0]), (0, c - w.shape[1])))


def _padb(b, d):
    return jnp.pad(b, (0, d - b.shape[0])).reshape(1, d)


def _cpar(v):
    return jnp.full((1, 16), v, jnp.int32)


def _tower(x, ei, batch, ones_tbl, w1, b1, w2, b2, w3, b3, relu_last):
    src2 = jnp.pad(ei[0], (0, EPAD - E)).reshape(EB, BLK)
    dst2 = jnp.pad(ei[1], (0, EPAD - E), constant_values=N).reshape(EB, BLK)
    bpad = jnp.pad(batch, (0, NPADG - N),
                   constant_values=G).reshape(NPADG // 128, 128)
    batch2 = jnp.stack([bpad, bpad])

    gix = _tc_index(src2).reshape(9, EPAD)
    dst1 = dst2.reshape(EPAD)

    w1p = _padw(w1, F, 48)
    w2p = _padw(w2, 48, 80)
    w3p = _padw(w3, 80, 144)
    b1p = _padb(b1, 48)
    b2p = _padb(b2, 80)
    b3p = _padb(b3, 144)

    degm = _AGG(ones_tbl, gix, dst1, _cpar(1))
    z1, g1, dinv8, d28 = _tc_first(degm, x, w1p)
    s1 = _AGG(g1.reshape(9 * N, 16), gix, dst1, _cpar(3))
    z2, g2 = _tc_mid(48, 80, s1, z1, dinv8, d28, b1p, w2p)
    s2 = _AGG(g2.reshape(9 * N, 16), gix, dst1, _cpar(5))
    z3, g3 = _tc_mid(80, 144, s2, z2, dinv8, d28, b2p, w3p)
    s3 = _AGG(g3.reshape(9 * N, 16), gix, dst1, _cpar(9))
    h4cb = _tc_last(s3, z3, dinv8, d28, b3p, relu_last)
    return _GEP(h4cb, batch2)


def kernel(pro1_x, pro1_edge_index, pro1_batch, pro2_x, pro2_edge_index,
           pro2_batch, w1W1, w1B1, w1W2, w1B2, w1W3, w1B3, w1W4, w1B4,
           w1W5, w1B5, w2W1, w2B1, w2W2, w2B2, w2W3, w2B3, w2W4, w2B4,
           w2W5, w2B5, fc1W, fc1B, fc2W, fc2B, outW, outB):
    ones_tbl = jnp.ones((9 * N, 16), jnp.float32)
    s1, c1 = _tower(pro1_x, pro1_edge_index, pro1_batch, ones_tbl,
                    w1W1, w1B1, w1W2, w1B2, w1W3, w1B3, relu_last=False)
    s2, c2 = _tower(pro2_x, pro2_edge_index, pro2_batch, ones_tbl,
                    w2W1, w2B1, w2W2, w2B2, w2W3, w2B3, relu_last=True)
    return _tc_head(
        s1, c1, s2, c2,
        _padw(w1W4, 144, 1024), w1B4.reshape(1, -1), w1W5, w1B5.reshape(1, -1),
        _padw(w2W4, 144, 1024), w2B4.reshape(1, -1), w2W5, w2B5.reshape(1, -1),
        fc1W, fc1B.reshape(1, -1), fc2W, fc2B.reshape(1, -1),
        outW, outB.reshape(1, -1))


# speedup vs baseline: 2.2204x; 1.0345x over previous
import dataclasses

import jax
import jax.numpy as jnp
from jax import lax
from jax.experimental import pallas as pl
from jax.experimental.pallas import tpu as pltpu
from jax.experimental.pallas import tpu_sc as plsc

N = 100000
E = 1600000
G = 256
F = 33

NC = 2
NS = 16
NW = NC * NS

EPAD = 1638400
EPW = EPAD // NW
BLK = 256
EB = EPAD // BLK
EBW = EPW // BLK
CHB = 20
NCH = EBW // CHB

ACC = 100352
RW = ACC // NS
ZR = 196

NPADG = 102400
GB = NPADG // NW
GACC = 272

_MESH = plsc.VectorSubcoreMesh(core_axis_name="c", subcore_axis_name="s")

TR = 1000
GRID = N // TR


def _init_const(ref, rows, val):
    @pl.loop(0, rows)
    def _(i):
        ref[i, :] = jnp.full((16,), val, jnp.float32)


def _agg_body(tbl_hbm, gix_hbm, dst_hbm, cpar_hbm, out_hbm,
              gidxc, didxc, cbuf, rows, zbuf, acc, gsem, ssem):
    c = lax.axis_index("c")
    s = lax.axis_index("s")
    wid = c * NS + s

    pltpu.sync_copy(cpar_hbm, cbuf)
    cval = jnp.max(cbuf[0, :])
    _init_const(zbuf, ZR, 0.0)

    def fire_gather(b, q):
        pltpu.async_copy(tbl_hbm.at[gidxc.at[pl.ds(b * BLK, BLK)]],
                         rows.at[q], gsem.at[q])

    def fire_scatter(b, q):
        pltpu.async_copy(rows.at[q], acc.at[didxc.at[pl.ds(b * BLK, BLK)]],
                         ssem.at[q], add=True)

    def drain(sem, q):
        pltpu.make_async_copy(tbl_hbm.at[pl.ds(0, BLK)], rows.at[q],
                              sem.at[q]).wait()

    @pl.loop(0, cval)
    def _(col):
        for k in range(RW // ZR):
            pltpu.sync_copy(zbuf, acc.at[pl.ds(s * RW + k * ZR, ZR)])
        plsc.subcore_barrier()

        @pl.loop(0, NCH)
        def _(ch):
            @pl.when(ch > 0)
            def _():
                for qq in range(4):
                    drain(ssem, (CHB - 4 + qq) % 4)

            ibase = (wid * EBW + ch * CHB) * BLK
            pltpu.sync_copy(gix_hbm.at[col, pl.ds(ibase, CHB * BLK)], gidxc)
            pltpu.sync_copy(dst_hbm.at[pl.ds(ibase, CHB * BLK)], didxc)

            for b in range(CHB):
                q = b % 4
                if b >= 2 and b + 2 < CHB:
                    drain(ssem, (b - 2) % 4)
                if b == 0:
                    fire_gather(0, 0)
                    fire_gather(1, 1)
                if b + 2 < CHB:
                    fire_gather(b + 2, (b + 2) % 4)
                drain(gsem, q)
                fire_scatter(b, q)

        for qq in range(4):
            drain(ssem, (CHB - 4 + qq) % 4)
        plsc.subcore_barrier()

        pltpu.sync_copy(acc.at[pl.ds(s * RW, RW)],
                        out_hbm.at[c, col, pl.ds(s * RW, RW)])


_SC_CP = pltpu.CompilerParams(use_tc_tiling_on_sc=False)
if "needs_layout_passes" in pltpu.CompilerParams.__dataclass_fields__:
    _SC_CP = dataclasses.replace(_SC_CP, needs_layout_passes=False)

_AGG = pl.kernel(
    _agg_body,
    out_type=jax.ShapeDtypeStruct((NC, 9, ACC, 16), jnp.float32),
    compiler_params=_SC_CP,
    mesh=_MESH,
    scratch_types=[
        pltpu.VMEM((CHB * BLK,), jnp.int32),
        pltpu.VMEM((CHB * BLK,), jnp.int32),
        pltpu.VMEM((1, 16), jnp.int32),
        pltpu.VMEM((4, BLK, 16), jnp.float32),
        pltpu.VMEM((ZR, 16), jnp.float32),
        pltpu.VMEM_SHARED((ACC, 16), jnp.float32),
        pltpu.SemaphoreType.DMA((4,)),
        pltpu.SemaphoreType.DMA((4,)),
    ],
)


def _gep_body(h_hbm, b_hbm, sums_hbm, counts_hbm, bidx, hbuf, zbuf, acc, ssem):
    c = lax.axis_index("c")
    s = lax.axis_index("s")
    wid = c * NS + s
    nr = GB // 128
    pltpu.sync_copy(b_hbm.at[c, pl.ds(wid * nr, nr)], bidx)
    _init_const(zbuf, GACC, 0.0)

    def scatter_all():
        cps = [
            pltpu.async_copy(hbuf.at[pl.ds(j * 128, 128)],
                             acc.at[bidx.at[j]], ssem, add=True)
            for j in range(nr)
        ]
        for cp in cps:
            cp.wait()

    _init_const(hbuf, GB, 1.0)

    @pl.when(s == 0)
    def _():
        pltpu.sync_copy(zbuf, acc)

    plsc.subcore_barrier()
    scatter_all()
    plsc.subcore_barrier()

    @pl.when(s == 0)
    def _():
        pltpu.sync_copy(acc, counts_hbm.at[c])

    @pl.loop(0, 9)
    def _(col):
        @pl.when(s == 0)
        def _():
            pltpu.sync_copy(zbuf, acc)

        plsc.subcore_barrier()
        pltpu.sync_copy(h_hbm.at[col, pl.ds(wid * GB, GB)], hbuf)
        scatter_all()
        plsc.subcore_barrier()

        @pl.when(s == 0)
        def _():
            pltpu.sync_copy(acc, sums_hbm.at[c, col])


_GEP = pl.kernel(
    _gep_body,
    out_type=[
        jax.ShapeDtypeStruct((NC, 9, GACC, 16), jnp.float32),
        jax.ShapeDtypeStruct((NC, GACC, 16), jnp.float32),
    ],
    compiler_params=_SC_CP,
    mesh=_MESH,
    scratch_types=[
        pltpu.VMEM((GB // 128, 128), jnp.int32),
        pltpu.VMEM((GB, 16), jnp.float32),
        pltpu.VMEM((GACC, 16), jnp.float32),
        pltpu.VMEM_SHARED((GACC, 16), jnp.float32),
        pltpu.SemaphoreType.DMA,
    ],
)


def _tc_index(src2):
    def bdy(s_ref, g_ref):
        sv = s_ref[...]
        for cc in range(9):
            g_ref[cc, :, :] = sv + cc * N

    eb = 16
    return pl.pallas_call(
        bdy,
        grid=(EB // eb,),
        in_specs=[pl.BlockSpec((eb, BLK), lambda i: (i, 0))],
        out_specs=[pl.BlockSpec((9, eb, BLK), lambda i: (0, i, 0))],
        out_shape=[jax.ShapeDtypeStruct((9, EB, BLK), jnp.int32)],
    )(src2)[0]


def _tc_first(degm, x, w1p):
    def bdy(deg_ref, x_ref, w_ref, z_ref, g_ref, dinv_ref, d2_ref):
        deg = deg_ref[0, 0, :, 0:1] + deg_ref[1, 0, :, 0:1] + 1.0
        dinv = lax.rsqrt(deg)
        z = jnp.dot(x_ref[...], w_ref[...], preferred_element_type=jnp.float32)
        g = z * dinv
        z_ref[...] = z
        for cc in range(3):
            g_ref[cc, :, :] = g[:, cc * 16:(cc + 1) * 16]
        dinv_ref[...] = jnp.broadcast_to(dinv, (TR, 8))
        d2_ref[...] = jnp.broadcast_to(dinv * dinv, (TR, 8))

    return pl.pallas_call(
        bdy,
        grid=(GRID,),
        in_specs=[
            pl.BlockSpec((NC, 1, TR, 16), lambda i: (0, 0, i, 0)),
            pl.BlockSpec((TR, F), lambda i: (i, 0)),
            pl.BlockSpec((F, 48), lambda i: (0, 0)),
        ],
        out_specs=[
            pl.BlockSpec((TR, 48), lambda i: (i, 0)),
            pl.BlockSpec((9, TR, 16), lambda i: (0, i, 0)),
            pl.BlockSpec((TR, 8), lambda i: (i, 0)),
            pl.BlockSpec((TR, 8), lambda i: (i, 0)),
        ],
        out_shape=[
            jax.ShapeDtypeStruct((N, 48), jnp.float32),
            jax.ShapeDtypeStruct((9, N, 16), jnp.float32),
            jax.ShapeDtypeStruct((N, 8), jnp.float32),
            jax.ShapeDtypeStruct((N, 8), jnp.float32),
        ],
    )(degm, x, w1p)


def _tc_mid(din, dout, sv, z, dinv8, d28, bp, wp):
    cn = dout // 16

    def bdy(s_ref, z_ref, dinv_ref, d2_ref, b_ref, w_ref, zo_ref, g_ref):
        dinv = dinv_ref[:, 0:1]
        d2 = d2_ref[:, 0:1]
        sv = jnp.concatenate([s_ref[0, cc] + s_ref[1, cc]
                              for cc in range(din // 16)], axis=1)
        a = jnp.maximum(dinv * sv + d2 * z_ref[...] + b_ref[...], 0.0)
        z2 = jnp.dot(a, w_ref[...], preferred_element_type=jnp.float32)
        g = z2 * dinv
        zo_ref[...] = z2
        for cc in range(cn):
            g_ref[cc, :, :] = g[:, cc * 16:(cc + 1) * 16]

    return pl.pallas_call(
        bdy,
        grid=(GRID,),
        in_specs=[
            pl.BlockSpec((NC, din // 16, TR, 16), lambda i: (0, 0, i, 0)),
            pl.BlockSpec((TR, din), lambda i: (i, 0)),
            pl.BlockSpec((TR, 8), lambda i: (i, 0)),
            pl.BlockSpec((TR, 8), lambda i: (i, 0)),
            pl.BlockSpec((1, din), lambda i: (0, 0)),
            pl.BlockSpec((din, dout), lambda i: (0, 0)),
        ],
        out_specs=[
            pl.BlockSpec((TR, dout), lambda i: (i, 0)),
            pl.BlockSpec((9, TR, 16), lambda i: (0, i, 0)),
        ],
        out_shape=[
            jax.ShapeDtypeStruct((N, dout), jnp.float32),
            jax.ShapeDtypeStruct((9, N, 16), jnp.float32),
        ],
    )(sv, z, dinv8, d28, bp, wp)


def _tc_last(sv, z, dinv8, d28, bp, relu_last):
    def bdy(s_ref, z_ref, dinv_ref, d2_ref, b_ref, h_ref):
        dinv = dinv_ref[:, 0:1]
        d2 = d2_ref[:, 0:1]
        sv = jnp.concatenate([s_ref[0, cc] + s_ref[1, cc]
                              for cc in range(9)], axis=1)
        a = dinv * sv + d2 * z_ref[...] + b_ref[...]
        if relu_last:
            a = jnp.maximum(a, 0.0)
        for cc in range(9):
            h_ref[cc, :, :] = a[:, cc * 16:(cc + 1) * 16]

    return pl.pallas_call(
        bdy,
        grid=(GRID,),
        in_specs=[
            pl.BlockSpec((NC, 9, TR, 16), lambda i: (0, 0, i, 0)),
            pl.BlockSpec((TR, 144), lambda i: (i, 0)),
            pl.BlockSpec((TR, 8), lambda i: (i, 0)),
            pl.BlockSpec((TR, 8), lambda i: (i, 0)),
            pl.BlockSpec((1, 144), lambda i: (0, 0)),
        ],
        out_specs=[pl.BlockSpec((9, TR, 16), lambda i: (0, i, 0))],
        out_shape=[jax.ShapeDtypeStruct((9, NPADG, 16), jnp.float32)],
    )(sv, z, dinv8, d28, bp)[0]


def _tc_head(s1, c1, s2, c2, w14, b14, w15, b15, w24, b24, w25, b25,
             fc1w, fc1b, fc2w, fc2b, outw, outb):
    def bdy(s1_ref, c1_ref, s2_ref, c2_ref, w14_ref, b14_ref, w15_ref, b15_ref,
            w24_ref, b24_ref, w25_ref, b25_ref, fc1w_ref, fc1b_ref,
            fc2w_ref, fc2b_ref, outw_ref, outb_ref, o_ref):
        def pool(sref, cref):
            svv = jnp.concatenate(
                [sref[0, cc] + sref[1, cc] for cc in range(9)], axis=1)
            cvv = cref[0, :, 0:1] + cref[1, :, 0:1]
            return (svv / jnp.maximum(cvv, 1.0))[:G]

        p1 = pool(s1_ref, c1_ref)
        p2 = pool(s2_ref, c2_ref)
        x = jnp.maximum(jnp.dot(p1, w14_ref[...],
                                preferred_element_type=jnp.float32)
                        + b14_ref[...], 0.0)
        x = jnp.dot(x, w15_ref[...],
                    preferred_element_type=jnp.float32) + b15_ref[...]
        y = jnp.maximum(jnp.dot(p2, w24_ref[...],
                                preferred_element_type=jnp.float32)
                        + b24_ref[...], 0.0)
        y = jnp.dot(y, w25_ref[...],
                    preferred_element_type=jnp.float32) + b25_ref[...]
        xc = jnp.concatenate([x, y], axis=1)
        xc = jnp.maximum(jnp.dot(xc, fc1w_ref[...],
                                 preferred_element_type=jnp.float32)
                         + fc1b_ref[...], 0.0)
        xc = jnp.maximum(jnp.dot(xc, fc2w_ref[...],
                                 preferred_element_type=jnp.float32)
                         + fc2b_ref[...], 0.0)
        o_ref[...] = jax.nn.sigmoid(
            jnp.dot(xc, outw_ref[...], preferred_element_type=jnp.float32)
            + outb_ref[...])

    return pl.pallas_call(
        bdy,
        out_shape=jax.ShapeDtypeStruct((G, 1), jnp.float32),
    )(s1, c1, s2, c2, w14, b14, w15, b15, w24, b24, w25, b25,
      fc1w, fc1b, fc2w, fc2b, outw, outb)


def _padw(w, r, c):
    return jnp.pad(w, ((0, r - w.shape[0]), (0, c - w.shape[1])))


def _padb(b, d):
    return jnp.pad(b, (0, d - b.shape[0])).reshape(1, d)


def _cpar(v):
    return jnp.full((1, 16), v, jnp.int32)


def _tower(x, ei, batch, ones_tbl, w1, b1, w2, b2, w3, b3, relu_last):
    src2 = jnp.pad(ei[0], (0, EPAD - E)).reshape(EB, BLK)
    dst2 = jnp.pad(ei[1], (0, EPAD - E), constant_values=N).reshape(EB, BLK)
    bpad = jnp.pad(batch, (0, NPADG - N),
                   constant_values=G).reshape(NPADG // 128, 128)
    batch2 = jnp.stack([bpad, bpad])

    gix = _tc_index(src2).reshape(9, EPAD)
    dst1 = dst2.reshape(EPAD)

    w1p = _padw(w1, F, 48)
    w2p = _padw(w2, 48, 80)
    w3p = _padw(w3, 80, 144)
    b1p = _padb(b1, 48)
    b2p = _padb(b2, 80)
    b3p = _padb(b3, 144)

    degm = _AGG(ones_tbl, gix, dst1, _cpar(1))
    z1, g1, dinv8, d28 = _tc_first(degm, x, w1p)
    s1 = _AGG(g1.reshape(9 * N, 16), gix, dst1, _cpar(3))
    z2, g2 = _tc_mid(48, 80, s1, z1, dinv8, d28, b1p, w2p)
    s2 = _AGG(g2.reshape(9 * N, 16), gix, dst1, _cpar(5))
    z3, g3 = _tc_mid(80, 144, s2, z2, dinv8, d28, b2p, w3p)
    s3 = _AGG(g3.reshape(9 * N, 16), gix, dst1, _cpar(9))
    h4cb = _tc_last(s3, z3, dinv8, d28, b3p, relu_last)
    return _GEP(h4cb, batch2)


def kernel(pro1_x, pro1_edge_index, pro1_batch, pro2_x, pro2_edge_index,
           pro2_batch, w1W1, w1B1, w1W2, w1B2, w1W3, w1B3, w1W4, w1B4,
           w1W5, w1B5, w2W1, w2B1, w2W2, w2B2, w2W3, w2B3, w2W4, w2B4,
           w2W5, w2B5, fc1W, fc1B, fc2W, fc2B, outW, outB):
    ones_tbl = jnp.ones((9 * N, 16), jnp.float32)
    s1, c1 = _tower(pro1_x, pro1_edge_index, pro1_batch, ones_tbl,
                    w1W1, w1B1, w1W2, w1B2, w1W3, w1B3, relu_last=False)
    s2, c2 = _tower(pro2_x, pro2_edge_index, pro2_batch, ones_tbl,
                    w2W1, w2B1, w2W2, w2B2, w2W3, w2B3, relu_last=True)
    return _tc_head(
        s1, c1, s2, c2,
        _padw(w1W4, 144, 1024), w1B4.reshape(1, -1), w1W5, w1B5.reshape(1, -1),
        _padw(w2W4, 144, 1024), w2B4.reshape(1, -1), w2W5, w2B5.reshape(1, -1),
        fc1W, fc1B.reshape(1, -1), fc2W, fc2B.reshape(1, -1),
        outW, outB.reshape(1, -1))
